# baseline (device time: 300887 ns/iter reference)
import jax
import jax.numpy as jnp
from jax import lax
from jax.experimental import pallas as pl
from jax.experimental.pallas import tpu as pltpu

N_DEV = 4


def kernel(x, router_W, route_idx, expert_W):
    n_tok, d = x.shape
    e_per, _, h = expert_W.shape

    def body(x_ref, rw_ref, ridx_ref, w_ref, out_ref, comm_ref, send_sems, recv_sems):
        my = lax.axis_index("i")
        left = lax.rem(my + N_DEV - 1, N_DEV)
        right = lax.rem(my + 1, N_DEV)

        barrier_sem = pltpu.get_barrier_semaphore()
        for nbr in (left, right):
            pl.semaphore_signal(
                barrier_sem, inc=1,
                device_id=(nbr,), device_id_type=pl.DeviceIdType.MESH,
            )
        pl.semaphore_wait(barrier_sem, 2)

        ridx = ridx_ref[:, 0]
        xv = x_ref[...]
        acc = jnp.zeros((n_tok, h), jnp.float32)
        for el in range(e_per):
            ge = my * e_per + el
            xm = jnp.where((ridx == ge)[:, None], xv, 0.0)
            acc = acc + jnp.dot(xm, w_ref[el], preferred_element_type=jnp.float32)
        out_ref[...] = acc
        comm_ref[0] = acc

        for hop in range(N_DEV - 1):
            s = hop % 2
            r = (hop + 1) % 2
            rdma = pltpu.make_async_remote_copy(
                src_ref=comm_ref.at[s],
                dst_ref=comm_ref.at[r],
                send_sem=send_sems.at[s],
                recv_sem=recv_sems.at[r],
                device_id=(right,),
                device_id_type=pl.DeviceIdType.MESH,
            )
            rdma.start()
            rdma.wait()
            out_ref[...] += comm_ref[r]

    return pl.pallas_call(
        body,
        out_shape=jax.ShapeDtypeStruct((n_tok, h), jnp.float32),
        in_specs=[
            pl.BlockSpec(memory_space=pltpu.VMEM),
            pl.BlockSpec(memory_space=pltpu.VMEM),
            pl.BlockSpec(memory_space=pltpu.VMEM),
            pl.BlockSpec(memory_space=pltpu.VMEM),
        ],
        out_specs=pl.BlockSpec(memory_space=pltpu.VMEM),
        scratch_shapes=[
            pltpu.VMEM((2, n_tok, h), jnp.float32),
            pltpu.SemaphoreType.DMA((2,)),
            pltpu.SemaphoreType.DMA((2,)),
        ],
        compiler_params=pltpu.CompilerParams(collective_id=0),
    )(x, router_W, route_idx, expert_W)


# device time: 101802 ns/iter; 2.9556x vs baseline; 2.9556x over previous
import jax
import jax.numpy as jnp
from jax import lax
from jax.experimental import pallas as pl
from jax.experimental.pallas import tpu as pltpu

N_DEV = 4


def kernel(x, router_W, route_idx, expert_W):
    n_tok, d = x.shape
    e_per, _, h = expert_W.shape
    C = n_tok // N_DEV
    HH = h // 2

    def body(x_ref, rw_ref, ridx_ref, w_ref, out_ref,
             part_ref, commR, commL, sendR, recvR, sendL, recvL):
        my = lax.axis_index("i")
        left = lax.rem(my + N_DEV - 1, N_DEV)
        right = lax.rem(my + 1, N_DEV)

        barrier_sem = pltpu.get_barrier_semaphore()
        for nbr in (left, right):
            pl.semaphore_signal(
                barrier_sem, inc=1,
                device_id=(nbr,), device_id_type=pl.DeviceIdType.MESH,
            )
        pl.semaphore_wait(barrier_sem, 2)

        ridx = ridx_ref[:, 0]
        xv = x_ref[...]
        acc = jnp.zeros((n_tok, h), jnp.float32)
        for el in range(e_per):
            ge = my * e_per + el
            xm = jnp.where((ridx == ge)[:, None], xv, 0.0)
            acc = acc + jnp.dot(xm, w_ref[el], preferred_element_type=jnp.float32)
        part_ref[...] = acc

        def pchunk(c, col0):
            return part_ref[pl.ds(c * C, C), pl.ds(col0, HH)]

        def chunk_at(offs):
            return lax.rem(my + offs + 4 * N_DEV, N_DEV)

        dirs = (
            (right, -1, 0, commR, sendR, recvR),
            (left, +1, HH, commL, sendL, recvL),
        )

        for (nbr, sg, col0, comm, ssem, rsem) in dirs:
            comm[0] = pchunk(chunk_at(sg * 1), col0)

        for s in range(N_DEV - 1):
            rdmas = []
            for (nbr, sg, col0, comm, ssem, rsem) in dirs:
                rdma = pltpu.make_async_remote_copy(
                    src_ref=comm.at[s % 2],
                    dst_ref=comm.at[(s + 1) % 2],
                    send_sem=ssem.at[s % 2],
                    recv_sem=rsem.at[(s + 1) % 2],
                    device_id=(nbr,),
                    device_id_type=pl.DeviceIdType.MESH,
                )
                rdma.start()
                rdmas.append(rdma)
            for rdma, (nbr, sg, col0, comm, ssem, rsem) in zip(rdmas, dirs):
                rdma.wait()
                comm[(s + 1) % 2] += pchunk(chunk_at(sg * (2 + s)), col0)

        for (nbr, sg, col0, comm, ssem, rsem) in dirs:
            out_ref[pl.ds(my * C, C), pl.ds(col0, HH)] = comm[1]

        for t in range(N_DEV - 1):
            rdmas = []
            for (nbr, sg, col0, comm, ssem, rsem) in dirs:
                rdma = pltpu.make_async_remote_copy(
                    src_ref=comm.at[(t + 1) % 2],
                    dst_ref=comm.at[t % 2],
                    send_sem=ssem.at[(t + 1) % 2],
                    recv_sem=rsem.at[t % 2],
                    device_id=(nbr,),
                    device_id_type=pl.DeviceIdType.MESH,
                )
                rdma.start()
                rdmas.append(rdma)
            for rdma, (nbr, sg, col0, comm, ssem, rsem) in zip(rdmas, dirs):
                rdma.wait()
                cr = chunk_at(sg * (t + 1))
                out_ref[pl.ds(cr * C, C), pl.ds(col0, HH)] = comm[t % 2]

    return pl.pallas_call(
        body,
        out_shape=jax.ShapeDtypeStruct((n_tok, h), jnp.float32),
        in_specs=[
            pl.BlockSpec(memory_space=pltpu.VMEM),
            pl.BlockSpec(memory_space=pltpu.VMEM),
            pl.BlockSpec(memory_space=pltpu.VMEM),
            pl.BlockSpec(memory_space=pltpu.VMEM),
        ],
        out_specs=pl.BlockSpec(memory_space=pltpu.VMEM),
        scratch_shapes=[
            pltpu.VMEM((n_tok, h), jnp.float32),
            pltpu.VMEM((2, C, HH), jnp.float32),
            pltpu.VMEM((2, C, HH), jnp.float32),
            pltpu.SemaphoreType.DMA((2,)),
            pltpu.SemaphoreType.DMA((2,)),
            pltpu.SemaphoreType.DMA((2,)),
            pltpu.SemaphoreType.DMA((2,)),
        ],
        compiler_params=pltpu.CompilerParams(collective_id=0),
    )(x, router_W, route_idx, expert_W)


# device time: 94706 ns/iter; 3.1771x vs baseline; 1.0749x over previous
import jax
import jax.numpy as jnp
from jax import lax
from jax.experimental import pallas as pl
from jax.experimental.pallas import tpu as pltpu

N_DEV = 4


def kernel(x, router_W, route_idx, expert_W):
    n_tok, d = x.shape
    e_per, _, h = expert_W.shape
    C = n_tok // N_DEV
    HH = h // 2

    def body(x_ref, rw_ref, ridx_ref, w_ref, out_ref,
             commR, commL, sendR, recvR, sendL, recvL):
        my = lax.axis_index("i")
        left = lax.rem(my + N_DEV - 1, N_DEV)
        right = lax.rem(my + 1, N_DEV)

        barrier_sem = pltpu.get_barrier_semaphore()
        for nbr in (left, right):
            pl.semaphore_signal(
                barrier_sem, inc=1,
                device_id=(nbr,), device_id_type=pl.DeviceIdType.MESH,
            )
        pl.semaphore_wait(barrier_sem, 2)

        def pchunk(c, col0):
            xc = x_ref[pl.ds(c * C, C), :]
            rc = ridx_ref[pl.ds(c * C, C), 0]
            acc = jnp.zeros((C, HH), jnp.float32)
            for el in range(e_per):
                ge = my * e_per + el
                xm = jnp.where((rc == ge)[:, None], xc, 0.0)
                acc = acc + jnp.dot(
                    xm, w_ref[el, :, col0:col0 + HH],
                    preferred_element_type=jnp.float32,
                )
            return acc

        def chunk_at(offs):
            return lax.rem(my + offs + 4 * N_DEV, N_DEV)

        dirs = (
            (right, -1, 0, commR, sendR, recvR),
            (left, +1, HH, commL, sendL, recvL),
        )

        for (nbr, sg, col0, comm, ssem, rsem) in dirs:
            comm[0] = pchunk(chunk_at(sg * 1), col0)

        for s in range(N_DEV - 1):
            rdmas = []
            for (nbr, sg, col0, comm, ssem, rsem) in dirs:
                rdma = pltpu.make_async_remote_copy(
                    src_ref=comm.at[s % 2],
                    dst_ref=comm.at[(s + 1) % 2],
                    send_sem=ssem.at[s % 2],
                    recv_sem=rsem.at[(s + 1) % 2],
                    device_id=(nbr,),
                    device_id_type=pl.DeviceIdType.MESH,
                )
                rdma.start()
                rdmas.append(rdma)
            parts = [
                pchunk(chunk_at(sg * (2 + s)), col0)
                for (nbr, sg, col0, comm, ssem, rsem) in dirs
            ]
            for rdma, part, (nbr, sg, col0, comm, ssem, rsem) in zip(
                rdmas, parts, dirs
            ):
                rdma.wait()
                comm[(s + 1) % 2] += part

        for (nbr, sg, col0, comm, ssem, rsem) in dirs:
            out_ref[pl.ds(my * C, C), pl.ds(col0, HH)] = comm[1]

        for t in range(N_DEV - 1):
            rdmas = []
            for (nbr, sg, col0, comm, ssem, rsem) in dirs:
                rdma = pltpu.make_async_remote_copy(
                    src_ref=comm.at[(t + 1) % 2],
                    dst_ref=comm.at[t % 2],
                    send_sem=ssem.at[(t + 1) % 2],
                    recv_sem=rsem.at[t % 2],
                    device_id=(nbr,),
                    device_id_type=pl.DeviceIdType.MESH,
                )
                rdma.start()
                rdmas.append(rdma)
            for rdma, (nbr, sg, col0, comm, ssem, rsem) in zip(rdmas, dirs):
                rdma.wait()
                cr = chunk_at(sg * (t + 1))
                out_ref[pl.ds(cr * C, C), pl.ds(col0, HH)] = comm[t % 2]

    return pl.pallas_call(
        body,
        out_shape=jax.ShapeDtypeStruct((n_tok, h), jnp.float32),
        in_specs=[
            pl.BlockSpec(memory_space=pltpu.VMEM),
            pl.BlockSpec(memory_space=pltpu.VMEM),
            pl.BlockSpec(memory_space=pltpu.VMEM),
            pl.BlockSpec(memory_space=pltpu.VMEM),
        ],
        out_specs=pl.BlockSpec(memory_space=pltpu.VMEM),
        scratch_shapes=[
            pltpu.VMEM((2, C, HH), jnp.float32),
            pltpu.VMEM((2, C, HH), jnp.float32),
            pltpu.SemaphoreType.DMA((2,)),
            pltpu.SemaphoreType.DMA((2,)),
            pltpu.SemaphoreType.DMA((2,)),
            pltpu.SemaphoreType.DMA((2,)),
        ],
        compiler_params=pltpu.CompilerParams(collective_id=0),
    )(x, router_W, route_idx, expert_W)


# device time: 85722 ns/iter; 3.5100x vs baseline; 1.1048x over previous
import jax
import jax.numpy as jnp
from jax import lax
from jax.experimental import pallas as pl
from jax.experimental.pallas import tpu as pltpu

N_DEV = 4
N_ROUND = 2 * (N_DEV - 1)


def kernel(x, router_W, route_idx, expert_W):
    n_tok, d = x.shape
    e_per, _, h = expert_W.shape
    C = n_tok // N_DEV
    Q = h // 4

    def body(x_ref, rw_ref, ridx_ref, w_ref, out_ref,
             comm0, comm1, comm2, comm3,
             ssem0, rsem0, ssem1, rsem1, ssem2, rsem2, ssem3, rsem3):
        my = lax.axis_index("i")
        left = lax.rem(my + N_DEV - 1, N_DEV)
        right = lax.rem(my + 1, N_DEV)

        barrier_sem = pltpu.get_barrier_semaphore()
        for nbr in (left, right):
            pl.semaphore_signal(
                barrier_sem, inc=1,
                device_id=(nbr,), device_id_type=pl.DeviceIdType.MESH,
            )
        pl.semaphore_wait(barrier_sem, 2)

        def pchunk(c, col0):
            xc = x_ref[pl.ds(c * C, C), :]
            rc = ridx_ref[pl.ds(c * C, C), 0]
            acc = jnp.zeros((C, Q), jnp.float32)
            for el in range(e_per):
                ge = my * e_per + el
                xm = jnp.where((rc == ge)[:, None], xc, 0.0)
                acc = acc + jnp.dot(
                    xm, w_ref[el, :, col0:col0 + Q],
                    preferred_element_type=jnp.float32,
                )
            return acc

        def chunk_at(offs):
            return lax.rem(my + offs + 4 * N_DEV, N_DEV)

        rings = (
            (right, -1, 0 * Q, comm0, ssem0, rsem0),
            (left, +1, 2 * Q, comm1, ssem1, rsem1),
            (right, -1, 1 * Q, comm2, ssem2, rsem2),
            (left, +1, 3 * Q, comm3, ssem3, rsem3),
        )

        def mk_rdma(ring, r):
            nbr, sg, col0, comm, ssem, rsem = ring
            return pltpu.make_async_remote_copy(
                src_ref=comm.at[r % 2],
                dst_ref=comm.at[(r + 1) % 2],
                send_sem=ssem.at[r % 2],
                recv_sem=rsem.at[(r + 1) % 2],
                device_id=(nbr,),
                device_id_type=pl.DeviceIdType.MESH,
            )

        pending = []
        for ring in rings:
            nbr, sg, col0, comm, ssem, rsem = ring
            comm[0] = pchunk(chunk_at(sg * 1), col0)
            rdma = mk_rdma(ring, 0)
            rdma.start()
            pending.append(rdma)

        for r in range(N_ROUND):
            for idx, ring in enumerate(rings):
                nbr, sg, col0, comm, ssem, rsem = ring
                if r < N_DEV - 1:
                    part = pchunk(chunk_at(sg * (2 + r)), col0)
                pending[idx].wait()
                rslot = (r + 1) % 2
                if r < N_DEV - 2:
                    comm[rslot] += part
                elif r == N_DEV - 2:
                    comm[rslot] += part
                    out_ref[pl.ds(my * C, C), pl.ds(col0, Q)] = comm[rslot]
                else:
                    cr = chunk_at(sg * (r - N_DEV + 2))
                    out_ref[pl.ds(cr * C, C), pl.ds(col0, Q)] = comm[rslot]
                if r < N_ROUND - 1:
                    nxt = mk_rdma(ring, r + 1)
                    nxt.start()
                    pending[idx] = nxt

    return pl.pallas_call(
        body,
        out_shape=jax.ShapeDtypeStruct((n_tok, h), jnp.float32),
        in_specs=[
            pl.BlockSpec(memory_space=pltpu.VMEM),
            pl.BlockSpec(memory_space=pltpu.VMEM),
            pl.BlockSpec(memory_space=pltpu.VMEM),
            pl.BlockSpec(memory_space=pltpu.VMEM),
        ],
        out_specs=pl.BlockSpec(memory_space=pltpu.VMEM),
        scratch_shapes=(
            [pltpu.VMEM((2, C, Q), jnp.float32)] * 4
            + [pltpu.SemaphoreType.DMA((2,))] * 8
        ),
        compiler_params=pltpu.CompilerParams(collective_id=0),
    )(x, router_W, route_idx, expert_W)


# device time: 85667 ns/iter; 3.5123x vs baseline; 1.0006x over previous
import jax
import jax.numpy as jnp
from jax import lax
from jax.experimental import pallas as pl
from jax.experimental.pallas import tpu as pltpu

N_DEV = 4
N_ROUND = 2 * (N_DEV - 1)
NSUB = 2


def kernel(x, router_W, route_idx, expert_W):
    n_tok, d = x.shape
    e_per, _, h = expert_W.shape
    C = n_tok // N_DEV
    NR = 2 * NSUB
    Q = h // NR

    def body(x_ref, rw_ref, ridx_ref, w_ref, out_ref, *scr):
        comms = scr[:NR]
        ssems = scr[NR:2 * NR]
        rsems = scr[2 * NR:]

        my = lax.axis_index("i")
        left = lax.rem(my + N_DEV - 1, N_DEV)
        right = lax.rem(my + 1, N_DEV)

        barrier_sem = pltpu.get_barrier_semaphore()
        for nbr in (left, right):
            pl.semaphore_signal(
                barrier_sem, inc=1,
                device_id=(nbr,), device_id_type=pl.DeviceIdType.MESH,
            )
        pl.semaphore_wait(barrier_sem, 2)

        def pchunk(c, col0):
            xc = x_ref[pl.ds(c * C, C), :]
            rc = ridx_ref[pl.ds(c * C, C), 0]
            acc = jnp.zeros((C, Q), jnp.float32)
            for el in range(e_per):
                ge = my * e_per + el
                xm = jnp.where((rc == ge)[:, None], xc, 0.0)
                acc = acc + jnp.dot(
                    xm, w_ref[el, :, col0:col0 + Q],
                    preferred_element_type=jnp.float32,
                )
            return acc

        def chunk_at(offs):
            return lax.rem(my + offs + 4 * N_DEV, N_DEV)

        rings = []
        for j in range(NSUB):
            rings.append((right, -1, j * Q, comms[2 * j], ssems[2 * j],
                          rsems[2 * j]))
            rings.append((left, +1, (NSUB + j) * Q, comms[2 * j + 1],
                          ssems[2 * j + 1], rsems[2 * j + 1]))

        def mk_rdma(ring, r):
            nbr, sg, col0, comm, ssem, rsem = ring
            if r < N_DEV - 1:
                src = comm.at[r % 2]
                dst = comm.at[(r + 1) % 2]
            else:
                cs = chunk_at(sg * (r - (N_DEV - 1)))
                sl_rows = pl.ds(cs * C, C)
                sl_cols = pl.ds(col0, Q)
                src = (comm.at[(N_DEV - 1) % 2] if r == N_DEV - 1
                       else out_ref.at[sl_rows, sl_cols])
                dst = out_ref.at[sl_rows, sl_cols]
            return pltpu.make_async_remote_copy(
                src_ref=src,
                dst_ref=dst,
                send_sem=ssem.at[r % 2],
                recv_sem=rsem.at[(r + 1) % 2],
                device_id=(nbr,),
                device_id_type=pl.DeviceIdType.MESH,
            )

        pending = []
        for ring in rings:
            nbr, sg, col0, comm, ssem, rsem = ring
            comm[0] = pchunk(chunk_at(sg * 1), col0)
            rdma = mk_rdma(ring, 0)
            rdma.start()
            pending.append(rdma)

        for r in range(N_ROUND):
            for idx, ring in enumerate(rings):
                nbr, sg, col0, comm, ssem, rsem = ring
                if r < N_DEV - 1:
                    part = pchunk(chunk_at(sg * (2 + r)), col0)
                pending[idx].wait()
                if r < N_DEV - 2:
                    comm[(r + 1) % 2] += part
                elif r == N_DEV - 2:
                    comm[(r + 1) % 2] += part
                    out_ref[pl.ds(my * C, C), pl.ds(col0, Q)] = (
                        comm[(r + 1) % 2]
                    )
                if r < N_ROUND - 1:
                    nxt = mk_rdma(ring, r + 1)
                    nxt.start()
                    pending[idx] = nxt

    return pl.pallas_call(
        body,
        out_shape=jax.ShapeDtypeStruct((n_tok, h), jnp.float32),
        in_specs=[
            pl.BlockSpec(memory_space=pltpu.VMEM),
            pl.BlockSpec(memory_space=pltpu.VMEM),
            pl.BlockSpec(memory_space=pltpu.VMEM),
            pl.BlockSpec(memory_space=pltpu.VMEM),
        ],
        out_specs=pl.BlockSpec(memory_space=pltpu.VMEM),
        scratch_shapes=(
            [pltpu.VMEM((2, C, Q), jnp.float32)] * NR
            + [pltpu.SemaphoreType.DMA((2,))] * NR
            + [pltpu.SemaphoreType.DMA((2,))] * NR
        ),
        compiler_params=pltpu.CompilerParams(collective_id=0),
    )(x, router_W, route_idx, expert_W)
